# baseline (speedup 1.0000x reference)
import functools

import jax
import jax.numpy as jnp
from jax import lax
from jax.experimental import pallas as pl
from jax.experimental.pallas import tpu as pltpu
from jax.experimental.pallas import tpu_sc as plsc

_N = 4096
_TC_ROWS = 3072
_SC_ROWS = _N - _TC_ROWS
_NW = 32
_QPW = _SC_ROWS // _NW
_L = 16
_JC = _N // _L
_QB = 8
_UNROLL = 2
_TI = 256
_BIG = 3.0e38


def _sc_body(pc1_hbm, pc2_hbm, ef_hbm, cham_hbm, idx_hbm, colp_hbm,
             p2i_v, px_v, py_v, pz_v, q1_v, qe_v, wq_v,
             qx_v, qy_v, qz_v, colmin_v, cham_v, idx_v):
    wid = lax.axis_index("c") * 16 + lax.axis_index("s")
    base = _TC_ROWS + wid * _QPW

    pltpu.sync_copy(pc2_hbm.at[pl.ds(0, 3 * _N)], p2i_v)
    pltpu.sync_copy(pc1_hbm.at[pl.ds(3 * base, 3 * _QPW)], q1_v)
    pltpu.sync_copy(ef_hbm.at[pl.ds(3 * base, 3 * _QPW)], qe_v)

    iota16 = lax.broadcasted_iota(jnp.int32, (_L,), 0)
    stride3 = iota16 * 3

    for c in range(3 * _QPW // _L):
        s = pl.ds(c * _L, _L)
        wq_v[s] = q1_v[s] + qe_v[s]

    for c in range(_QPW // _L):
        s = pl.ds(c * _L, _L)
        idx = stride3 + (c * _L * 3)
        qx_v[s] = plsc.load_gather(wq_v, [idx])
        qy_v[s] = plsc.load_gather(wq_v, [idx + 1])
        qz_v[s] = plsc.load_gather(wq_v, [idx + 2])

    def deint_body(c, _):
        s = pl.ds(c * _L, _L)
        idx = stride3 + c * (_L * 3)
        px_v[s] = plsc.load_gather(p2i_v, [idx])
        py_v[s] = plsc.load_gather(p2i_v, [idx + 1])
        pz_v[s] = plsc.load_gather(p2i_v, [idx + 2])
        colmin_v[s] = jnp.full((_L,), _BIG, jnp.float32)
        return 0
    lax.fori_loop(0, _JC, deint_body, 0)

    def _bcast(vec, lane):
        idx = jnp.full((_L, 1), lane, jnp.int32)
        dn = lax.GatherDimensionNumbers(
            offset_dims=(), collapsed_slice_dims=(0,), start_index_map=(0,))
        return lax.gather(vec, idx, dn, (1,),
                          mode=lax.GatherScatterMode.PROMISE_IN_BOUNDS)

    chamacc = jnp.zeros((_L,), jnp.float32)
    idxacc = jnp.zeros((_L,), jnp.int32)

    for b in range(_QPW // _QB):
        qoff = (b * _QB // _L) * _L
        qx16 = qx_v[pl.ds(qoff, _L)]
        qy16 = qy_v[pl.ds(qoff, _L)]
        qz16 = qz_v[pl.ds(qoff, _L)]
        bxs, bys, bzs = [], [], []
        for l in range(_QB):
            lane = (b * _QB + l) % _L
            bxs.append(_bcast(qx16, lane))
            bys.append(_bcast(qy16, lane))
            bzs.append(_bcast(qz16, lane))

        def scan_body(jc, carry):
            rms, ris = carry
            rms, ris = list(rms), list(ris)
            for u in range(_UNROLL):
                jb = jc * (_L * _UNROLL) + u * _L
                px = px_v[pl.ds(jb, _L)]
                py = py_v[pl.ds(jb, _L)]
                pz = pz_v[pl.ds(jb, _L)]
                cm = colmin_v[pl.ds(jb, _L)]
                jv = iota16 + jb
                for l in range(_QB):
                    d = (jnp.abs(px - bxs[l]) + jnp.abs(py - bys[l])) \
                        + jnp.abs(pz - bzs[l])
                    cm = jnp.minimum(cm, d)
                    lt = d < rms[l]
                    rms[l] = jnp.where(lt, d, rms[l])
                    ris[l] = jnp.where(lt, jv, ris[l])
                colmin_v[pl.ds(jb, _L)] = cm
            return tuple(rms), tuple(ris)

        rms0 = tuple(jnp.full((_L,), _BIG, jnp.float32) for _ in range(_QB))
        ris0 = tuple(jnp.zeros((_L,), jnp.int32) for _ in range(_QB))
        rms, ris = lax.fori_loop(0, _JC // _UNROLL, scan_body, (rms0, ris0))

        for l in range(_QB):
            i = b * _QB + l
            m = jnp.min(rms[l])
            jm = jnp.min(jnp.where(rms[l] == m, ris[l], _N))
            lsel = iota16 == (i % _L)
            chamacc = jnp.where(lsel, m, chamacc)
            idxacc = jnp.where(lsel, jm, idxacc)
        if (b * _QB + _QB) % _L == 0:
            o = ((b * _QB + _QB) // _L - 1) * _L
            cham_v[pl.ds(o, _L)] = chamacc
            idx_v[pl.ds(o, _L)] = idxacc

    pltpu.sync_copy(cham_v, cham_hbm.at[pl.ds(wid * _QPW, _QPW)])
    pltpu.sync_copy(idx_v, idx_hbm.at[pl.ds(wid * _QPW, _QPW)])
    pltpu.sync_copy(colmin_v, colp_hbm.at[pl.ds(wid * _N, _N)])


def _tc_body(pc1_ref, ef_ref, p_ref, cham_ref, idx_ref, colmin_ref,
             rowsum_ref):
    step = pl.program_id(0)

    w = pc1_ref[...] + ef_ref[...]
    wx = w[:, 0:1]
    wy = w[:, 1:2]
    wz = w[:, 2:3]
    px = p_ref[0:1, :]
    py = p_ref[1:2, :]
    pz = p_ref[2:3, :]

    d = (jnp.abs(wx - px) + jnp.abs(wy - py)) + jnp.abs(wz - pz)

    rmin = jnp.min(d, axis=1, keepdims=True)
    jidx = lax.broadcasted_iota(jnp.int32, d.shape, 1)
    ridx = jnp.min(jnp.where(d == rmin, jidx, _N), axis=1, keepdims=True)
    cham_ref[...] = rmin
    idx_ref[...] = ridx

    cmin = jnp.min(d, axis=0, keepdims=True)

    @pl.when(step == 0)
    def _init():
        colmin_ref[...] = cmin
        rowsum_ref[...] = jnp.full((1, 1), jnp.sum(rmin), jnp.float32)

    @pl.when(step != 0)
    def _acc():
        colmin_ref[...] = jnp.minimum(colmin_ref[...], cmin)
        rowsum_ref[...] = rowsum_ref[...] + jnp.sum(rmin)


def _loss_body(chamtc_ref, idxtc_ref, chamsc_ref, idxsc_ref, colp_ref,
               colmin_tc_ref, rowsum_tc_ref,
               loss_ref, cham_ref, idx_ref):
    sum_x = rowsum_tc_ref[0, 0] + jnp.sum(chamsc_ref[...])
    colmin = jnp.minimum(jnp.min(colp_ref[...], axis=0, keepdims=True),
                         colmin_tc_ref[...])
    mean_y = jnp.sum(colmin) / _N
    loss_ref[...] = jnp.full((1, 1), (sum_x / _N + mean_y) * 0.5, jnp.float32)
    cham_ref[:, 0:_TC_ROWS] = chamtc_ref[...].reshape(1, _TC_ROWS)
    cham_ref[:, _TC_ROWS:_N] = chamsc_ref[...].reshape(1, _SC_ROWS)
    idx_ref[:, 0:_TC_ROWS] = idxtc_ref[...].reshape(1, _TC_ROWS)
    idx_ref[:, _TC_ROWS:_N] = idxsc_ref[...].reshape(1, _SC_ROWS)


@jax.jit
def _chamfer(pc1r, pc1f, pc2f, efr, eff, p2rows):
    sc = functools.partial(
        pl.kernel,
        mesh=plsc.VectorSubcoreMesh(core_axis_name="c", subcore_axis_name="s"),
        compiler_params=pltpu.CompilerParams(needs_layout_passes=False),
        out_type=[
            jax.ShapeDtypeStruct((_SC_ROWS,), jnp.float32),
            jax.ShapeDtypeStruct((_SC_ROWS,), jnp.int32),
            jax.ShapeDtypeStruct((_NW * _N,), jnp.float32),
        ],
        scratch_types=[
            pltpu.VMEM((3 * _N,), jnp.float32),
            pltpu.VMEM((_N,), jnp.float32),
            pltpu.VMEM((_N,), jnp.float32),
            pltpu.VMEM((_N,), jnp.float32),
            pltpu.VMEM((3 * _QPW,), jnp.float32),
            pltpu.VMEM((3 * _QPW,), jnp.float32),
            pltpu.VMEM((3 * _QPW,), jnp.float32),
            pltpu.VMEM((_QPW,), jnp.float32),
            pltpu.VMEM((_QPW,), jnp.float32),
            pltpu.VMEM((_QPW,), jnp.float32),
            pltpu.VMEM((_N,), jnp.float32),
            pltpu.VMEM((_QPW,), jnp.float32),
            pltpu.VMEM((_QPW,), jnp.int32),
        ],
    )(_sc_body)
    cham_sc, idx_sc, colp = sc(pc1f, pc2f, eff)

    cham_tc, idx_tc, colmin_tc, rowsum_tc = pl.pallas_call(
        _tc_body,
        grid=(_TC_ROWS // _TI,),
        in_specs=[
            pl.BlockSpec((_TI, 3), lambda i: (i, 0)),
            pl.BlockSpec((_TI, 3), lambda i: (i, 0)),
            pl.BlockSpec((8, _N), lambda i: (0, 0)),
        ],
        out_specs=[
            pl.BlockSpec((_TI, 1), lambda i: (i, 0)),
            pl.BlockSpec((_TI, 1), lambda i: (i, 0)),
            pl.BlockSpec((1, _N), lambda i: (0, 0)),
            pl.BlockSpec((1, 1), lambda i: (0, 0)),
        ],
        out_shape=[
            jax.ShapeDtypeStruct((_TC_ROWS, 1), jnp.float32),
            jax.ShapeDtypeStruct((_TC_ROWS, 1), jnp.int32),
            jax.ShapeDtypeStruct((1, _N), jnp.float32),
            jax.ShapeDtypeStruct((1, 1), jnp.float32),
        ],
    )(pc1r[:_TC_ROWS], efr[:_TC_ROWS], p2rows)

    loss, cham, idx = pl.pallas_call(
        _loss_body,
        in_specs=[
            pl.BlockSpec((_TC_ROWS, 1), lambda: (0, 0)),
            pl.BlockSpec((_TC_ROWS, 1), lambda: (0, 0)),
            pl.BlockSpec((8, _SC_ROWS // 8), lambda: (0, 0)),
            pl.BlockSpec((8, _SC_ROWS // 8), lambda: (0, 0)),
            pl.BlockSpec((_NW, _N), lambda: (0, 0)),
            pl.BlockSpec((1, _N), lambda: (0, 0)),
            pl.BlockSpec((1, 1), lambda: (0, 0)),
        ],
        out_specs=[
            pl.BlockSpec((1, 1), lambda: (0, 0)),
            pl.BlockSpec((1, _N), lambda: (0, 0)),
            pl.BlockSpec((1, _N), lambda: (0, 0)),
        ],
        out_shape=[
            jax.ShapeDtypeStruct((1, 1), jnp.float32),
            jax.ShapeDtypeStruct((1, _N), jnp.float32),
            jax.ShapeDtypeStruct((1, _N), jnp.int32),
        ],
    )(cham_tc, idx_tc, cham_sc.reshape(8, _SC_ROWS // 8),
      idx_sc.reshape(8, _SC_ROWS // 8), colp.reshape(_NW, _N),
      colmin_tc, rowsum_tc)
    return loss, cham, idx


def kernel(pc1, pc2, est_flow):
    pc1r = pc1.reshape(_N, 3)
    efr = est_flow.reshape(_N, 3)
    p2c = pc2.reshape(_N, 3).T
    p2rows = jnp.zeros((8, _N), jnp.float32).at[0:3, :].set(p2c)
    loss, cham, idx = _chamfer(pc1r, pc1.reshape(3 * _N),
                               pc2.reshape(3 * _N), efr,
                               est_flow.reshape(3 * _N), p2rows)
    return (loss[0, 0], cham, idx)


# speedup vs baseline: 1.0425x; 1.0425x over previous
import functools

import jax
import jax.numpy as jnp
from jax import lax
from jax.experimental import pallas as pl
from jax.experimental.pallas import tpu as pltpu
from jax.experimental.pallas import tpu_sc as plsc

_N = 4096
_TC_ROWS = 3072
_SC_ROWS = _N - _TC_ROWS
_NW = 32
_QPW = _SC_ROWS // _NW
_L = 16
_JC = _N // _L
_QB = 8
_UNROLL = 2
_TI = 256
_BIG = 3.0e38


def _sc_body(wp_hbm, cham_hbm, idx_hbm, colp_hbm, sums_hbm,
             px_v, py_v, pz_v, qx_v, qy_v, qz_v, colmin_v, cham_v, idx_v,
             sum_v):
    wid = lax.axis_index("c") * 16 + lax.axis_index("s")
    base = _TC_ROWS + wid * _QPW

    pltpu.sync_copy(wp_hbm.at[pl.ds(3 * _N, _N)], px_v)
    pltpu.sync_copy(wp_hbm.at[pl.ds(4 * _N, _N)], py_v)
    pltpu.sync_copy(wp_hbm.at[pl.ds(5 * _N, _N)], pz_v)
    pltpu.sync_copy(wp_hbm.at[pl.ds(base, _QPW)], qx_v)
    pltpu.sync_copy(wp_hbm.at[pl.ds(_N + base, _QPW)], qy_v)
    pltpu.sync_copy(wp_hbm.at[pl.ds(2 * _N + base, _QPW)], qz_v)

    iota16 = lax.broadcasted_iota(jnp.int32, (_L,), 0)

    def init_body(c, _):
        colmin_v[pl.ds(c * _L, _L)] = jnp.full((_L,), _BIG, jnp.float32)
        return 0
    lax.fori_loop(0, _JC, init_body, 0)

    def _bcast(vec, lane):
        idx = jnp.full((_L, 1), lane, jnp.int32)
        dn = lax.GatherDimensionNumbers(
            offset_dims=(), collapsed_slice_dims=(0,), start_index_map=(0,))
        return lax.gather(vec, idx, dn, (1,),
                          mode=lax.GatherScatterMode.PROMISE_IN_BOUNDS)

    chamacc = jnp.zeros((_L,), jnp.float32)
    idxacc = jnp.zeros((_L,), jnp.int32)

    for b in range(_QPW // _QB):
        qoff = (b * _QB // _L) * _L
        qx16 = qx_v[pl.ds(qoff, _L)]
        qy16 = qy_v[pl.ds(qoff, _L)]
        qz16 = qz_v[pl.ds(qoff, _L)]
        bxs, bys, bzs = [], [], []
        for l in range(_QB):
            lane = (b * _QB + l) % _L
            bxs.append(_bcast(qx16, lane))
            bys.append(_bcast(qy16, lane))
            bzs.append(_bcast(qz16, lane))

        def scan_body(jc, carry):
            rms, ris = carry
            rms, ris = list(rms), list(ris)
            for u in range(_UNROLL):
                jb = jc * (_L * _UNROLL) + u * _L
                px = px_v[pl.ds(jb, _L)]
                py = py_v[pl.ds(jb, _L)]
                pz = pz_v[pl.ds(jb, _L)]
                cm = colmin_v[pl.ds(jb, _L)]
                jv = iota16 + jb
                for l in range(_QB):
                    d = (jnp.abs(px - bxs[l]) + jnp.abs(py - bys[l])) \
                        + jnp.abs(pz - bzs[l])
                    cm = jnp.minimum(cm, d)
                    lt = d < rms[l]
                    rms[l] = jnp.where(lt, d, rms[l])
                    ris[l] = jnp.where(lt, jv, ris[l])
                colmin_v[pl.ds(jb, _L)] = cm
            return tuple(rms), tuple(ris)

        rms0 = tuple(jnp.full((_L,), _BIG, jnp.float32) for _ in range(_QB))
        ris0 = tuple(jnp.zeros((_L,), jnp.int32) for _ in range(_QB))
        rms, ris = lax.fori_loop(0, _JC // _UNROLL, scan_body, (rms0, ris0))

        for l in range(_QB):
            i = b * _QB + l
            m = jnp.min(rms[l])
            jm = jnp.min(jnp.where(rms[l] == m, ris[l], _N))
            lsel = iota16 == (i % _L)
            chamacc = jnp.where(lsel, m, chamacc)
            idxacc = jnp.where(lsel, jm, idxacc)
        if (b * _QB + _QB) % _L == 0:
            o = ((b * _QB + _QB) // _L - 1) * _L
            cham_v[pl.ds(o, _L)] = chamacc
            idx_v[pl.ds(o, _L)] = idxacc

    psum = jnp.zeros((_L,), jnp.float32)
    for c in range(_QPW // _L):
        psum = psum + cham_v[pl.ds(c * _L, _L)]
    sum_v[pl.ds(0, _L)] = psum

    pltpu.sync_copy(cham_v, cham_hbm.at[pl.ds(wid * _QPW, _QPW)])
    pltpu.sync_copy(idx_v, idx_hbm.at[pl.ds(wid * _QPW, _QPW)])
    pltpu.sync_copy(colmin_v, colp_hbm.at[pl.ds(wid * _N, _N)])
    pltpu.sync_copy(sum_v, sums_hbm.at[pl.ds(wid * _L, _L)])


def _tc_body(pc1_ref, ef_ref, p_ref, cham_ref, idx_ref, colmin_ref,
             rowsum_ref):
    step = pl.program_id(0)

    w = pc1_ref[...] + ef_ref[...]
    wx = w[:, 0:1]
    wy = w[:, 1:2]
    wz = w[:, 2:3]
    px = p_ref[0:1, :]
    py = p_ref[1:2, :]
    pz = p_ref[2:3, :]

    d = (jnp.abs(wx - px) + jnp.abs(wy - py)) + jnp.abs(wz - pz)

    rmin = jnp.min(d, axis=1, keepdims=True)
    jidx = lax.broadcasted_iota(jnp.int32, d.shape, 1)
    ridx = jnp.min(jnp.where(d == rmin, jidx, _N), axis=1, keepdims=True)
    cham_ref[...] = rmin
    idx_ref[...] = ridx

    cmin = jnp.min(d, axis=0, keepdims=True)

    @pl.when(step == 0)
    def _init():
        colmin_ref[...] = cmin
        rowsum_ref[...] = jnp.full((1, 1), jnp.sum(rmin), jnp.float32)

    @pl.when(step != 0)
    def _acc():
        colmin_ref[...] = jnp.minimum(colmin_ref[...], cmin)
        rowsum_ref[...] = rowsum_ref[...] + jnp.sum(rmin)


def _loss_body(sums_ref, colp_ref, colmin_tc_ref, rowsum_tc_ref, loss_ref):
    sum_x = rowsum_tc_ref[0, 0] + jnp.sum(sums_ref[...])
    colmin = jnp.minimum(jnp.min(colp_ref[...], axis=0, keepdims=True),
                         colmin_tc_ref[...])
    mean_y = jnp.sum(colmin) / _N
    loss_ref[...] = jnp.full((1, 1), (sum_x / _N + mean_y) * 0.5, jnp.float32)


@jax.jit
def _chamfer(pc1r, efr, p2rows, wp):
    sc = functools.partial(
        pl.kernel,
        mesh=plsc.VectorSubcoreMesh(core_axis_name="c", subcore_axis_name="s"),
        compiler_params=pltpu.CompilerParams(needs_layout_passes=False),
        out_type=[
            jax.ShapeDtypeStruct((_SC_ROWS,), jnp.float32),
            jax.ShapeDtypeStruct((_SC_ROWS,), jnp.int32),
            jax.ShapeDtypeStruct((_NW * _N,), jnp.float32),
            jax.ShapeDtypeStruct((_NW * _L,), jnp.float32),
        ],
        scratch_types=[
            pltpu.VMEM((_N,), jnp.float32),
            pltpu.VMEM((_N,), jnp.float32),
            pltpu.VMEM((_N,), jnp.float32),
            pltpu.VMEM((_QPW,), jnp.float32),
            pltpu.VMEM((_QPW,), jnp.float32),
            pltpu.VMEM((_QPW,), jnp.float32),
            pltpu.VMEM((_N,), jnp.float32),
            pltpu.VMEM((_QPW,), jnp.float32),
            pltpu.VMEM((_QPW,), jnp.int32),
            pltpu.VMEM((_L,), jnp.float32),
        ],
    )(_sc_body)
    cham_sc, idx_sc, colp, sums = sc(wp)

    cham_tc, idx_tc, colmin_tc, rowsum_tc = pl.pallas_call(
        _tc_body,
        grid=(_TC_ROWS // _TI,),
        in_specs=[
            pl.BlockSpec((_TI, 3), lambda i: (i, 0)),
            pl.BlockSpec((_TI, 3), lambda i: (i, 0)),
            pl.BlockSpec((8, _N), lambda i: (0, 0)),
        ],
        out_specs=[
            pl.BlockSpec((_TI, 1), lambda i: (i, 0)),
            pl.BlockSpec((_TI, 1), lambda i: (i, 0)),
            pl.BlockSpec((1, _N), lambda i: (0, 0)),
            pl.BlockSpec((1, 1), lambda i: (0, 0)),
        ],
        out_shape=[
            jax.ShapeDtypeStruct((_TC_ROWS, 1), jnp.float32),
            jax.ShapeDtypeStruct((_TC_ROWS, 1), jnp.int32),
            jax.ShapeDtypeStruct((1, _N), jnp.float32),
            jax.ShapeDtypeStruct((1, 1), jnp.float32),
        ],
    )(pc1r[:_TC_ROWS], efr[:_TC_ROWS], p2rows)

    loss = pl.pallas_call(
        _loss_body,
        in_specs=[
            pl.BlockSpec((8, (_NW * _L) // 8), lambda: (0, 0)),
            pl.BlockSpec((_NW, _N), lambda: (0, 0)),
            pl.BlockSpec((1, _N), lambda: (0, 0)),
            pl.BlockSpec((1, 1), lambda: (0, 0)),
        ],
        out_specs=pl.BlockSpec((1, 1), lambda: (0, 0)),
        out_shape=jax.ShapeDtypeStruct((1, 1), jnp.float32),
    )(sums.reshape(8, (_NW * _L) // 8), colp.reshape(_NW, _N),
      colmin_tc, rowsum_tc)

    cham = jnp.concatenate([cham_tc.reshape(_TC_ROWS), cham_sc])
    idx = jnp.concatenate([idx_tc.reshape(_TC_ROWS), idx_sc])
    return loss, cham, idx


def kernel(pc1, pc2, est_flow):
    pc1r = pc1.reshape(_N, 3)
    efr = est_flow.reshape(_N, 3)
    wqc = (pc1 + est_flow).reshape(_N, 3).T
    p2c = pc2.reshape(_N, 3).T
    p2rows = jnp.zeros((8, _N), jnp.float32).at[0:3, :].set(p2c)
    wp = jnp.concatenate([wqc.reshape(3 * _N), p2c.reshape(3 * _N)])
    loss, cham, idx = _chamfer(pc1r, efr, p2rows, wp)
    return (loss[0, 0], cham.reshape(1, _N), idx.reshape(1, _N))
